# baseline (speedup 1.0000x reference)
import jax
import jax.numpy as jnp
from jax import lax
from jax.experimental import pallas as pl
from jax.experimental.pallas import tpu as pltpu
from jax.experimental.pallas import tpu_sc as plsc

VOCAB = 49408
EMBED = 768
MAX_LEN = 77
BATCH = 1024
LPAD = 80

NC, NS = 2, 16
NW = NC * NS
EW = BATCH // NW
CR = 8
NFULL = MAX_LEN // CR
TAIL = MAX_LEN - NFULL * CR
SUB = EMBED // 16


def _sc_body(tok_hbm, posn_hbm, ttab_hbm, ptab_hbm, out_hbm,
             it0, it1, idx_p, ptab_v, bo, tb,
             si0, si1, so,
             c0, c1, c2, c3, c4, c5, c6, c7, c8, c9):
    it = (it0, it1)
    si = (si0, si1)
    cs = (c0, c1, c2, c3, c4, c5, c6, c7, c8, c9)

    wid = lax.axis_index("s") * NC + lax.axis_index("c")
    pltpu.sync_copy(posn_hbm.at[pl.ds(pl.multiple_of(wid * EW * LPAD, 8),
                                      EW * LPAD)],
                    idx_p.at[pl.ds(0, EW * LPAD)])
    pltpu.sync_copy(ptab_hbm, ptab_v)

    ent0 = wid * EW

    def issue_idx(e, par):
        off = pl.multiple_of((ent0 + e) * LPAD, 8)
        pltpu.async_copy(tok_hbm.at[pl.ds(off, LPAD)], it[par], si[par])

    def wait_idx(par):
        pltpu.make_async_copy(tok_hbm.at[pl.ds(0, LPAD)],
                              it[par], si[par]).wait()

    def issue_gathers(par):
        for j in range(NFULL):
            pltpu.async_copy(
                ttab_hbm.at[it[par].at[pl.ds(j * CR, CR)]],
                bo.at[pl.ds(j * CR, CR)], cs[j])
        pltpu.async_copy(
            ttab_hbm.at[it[par].at[pl.ds(NFULL * CR, TAIL)]], tb, cs[NFULL])

    def wait_gather(par, j):
        if j < NFULL:
            pltpu.make_async_copy(ttab_hbm.at[it[par].at[pl.ds(0, CR)]],
                                  bo.at[pl.ds(0, CR)], cs[j]).wait()
        else:
            pltpu.make_async_copy(ttab_hbm.at[it[par].at[pl.ds(0, TAIL)]],
                                  tb, cs[j]).wait()

    def issue_out(e):
        pltpu.async_copy(bo, out_hbm.at[ent0 + e], so)

    def wait_out():
        pltpu.make_async_copy(bo, out_hbm.at[0], so).wait()

    cols = [jnp.arange(16, dtype=jnp.int32) + 16 * g for g in range(SUB)]

    def compute_chunk(e, j):
        n = CR if j < NFULL else TAIL
        coff = pl.multiple_of(e * LPAD + j * CR, 8)
        p_vec = idx_p[pl.ds(coff, 16)]

        def row(r, carry):
            pb = p_vec.at[jnp.full((16,), r, dtype=jnp.int32)].get(
                mode="promise_in_bounds")
            for q in range(0, SUB, 8):
                pvs = [plsc.load_gather(ptab_v, [pb + cols[g]])
                       for g in range(q, q + 8)]
                for g in range(q, q + 8):
                    s = pl.ds(16 * g, 16)
                    if j < NFULL:
                        plsc.addupdate(bo.at[j * CR + r, s], pvs[g - q])
                    else:
                        bo[j * CR + r, s] = tb[r, s] + pvs[g - q]
            return carry

        lax.fori_loop(0, n, row, 0)

    issue_idx(0, 0)

    def entry(e, carry):
        for par in range(2):
            ee = 2 * e + par
            wait_idx(par)
            pl.when(ee >= 1)(wait_out)
            issue_gathers(par)
            pl.when(ee < EW - 1)(lambda: issue_idx(ee + 1, par ^ 1))
            for j in range(NFULL + 1):
                wait_gather(par, j)
                compute_chunk(ee, j)
            issue_out(ee)
        return carry

    lax.fori_loop(0, EW // 2, entry, 0)
    wait_out()


def kernel(tokens, positions, token_table, pos_table):
    pad = ((0, 0), (0, LPAD - MAX_LEN))
    tok = jnp.pad(tokens, pad).reshape(BATCH * LPAD)
    posn = jnp.pad(positions, pad).reshape(BATCH * LPAD) * EMBED
    ptab = pos_table.reshape(MAX_LEN * EMBED)
    mesh = plsc.VectorSubcoreMesh(
        core_axis_name="c", subcore_axis_name="s",
        num_cores=NC, num_subcores=NS)
    return pl.kernel(
        _sc_body,
        out_type=jax.ShapeDtypeStruct((BATCH, MAX_LEN, EMBED), jnp.float32),
        mesh=mesh,
        compiler_params=pltpu.CompilerParams(
            use_tc_tiling_on_sc=False, needs_layout_passes=False),
        scratch_types=[
            pltpu.VMEM((LPAD,), jnp.int32),
            pltpu.VMEM((LPAD,), jnp.int32),
            pltpu.VMEM((EW * LPAD + 16,), jnp.int32),
            pltpu.VMEM((MAX_LEN * EMBED,), jnp.float32),
            pltpu.VMEM((MAX_LEN, EMBED), jnp.float32),
            pltpu.VMEM((TAIL, EMBED), jnp.float32),
            pltpu.SemaphoreType.DMA,
            pltpu.SemaphoreType.DMA,
            pltpu.SemaphoreType.DMA,
            pltpu.SemaphoreType.DMA,
            pltpu.SemaphoreType.DMA,
            pltpu.SemaphoreType.DMA,
            pltpu.SemaphoreType.DMA,
            pltpu.SemaphoreType.DMA,
            pltpu.SemaphoreType.DMA,
            pltpu.SemaphoreType.DMA,
            pltpu.SemaphoreType.DMA,
            pltpu.SemaphoreType.DMA,
            pltpu.SemaphoreType.DMA,
        ],
    )(tok, posn, token_table, ptab)


# speedup vs baseline: 1.1117x; 1.1117x over previous
import jax
import jax.numpy as jnp
from jax import lax
from jax.experimental import pallas as pl
from jax.experimental.pallas import tpu as pltpu
from jax.experimental.pallas import tpu_sc as plsc

VOCAB = 49408
EMBED = 768
MAX_LEN = 77
BATCH = 1024

NC, NS = 2, 16
NW = NC * NS
ROWS = BATCH * MAX_LEN
WROWS = ROWS // NW
CR = 16
NCH = WROWS // CR
SUB = EMBED // 16


def _sc_body(tok_hbm, posn_hbm, ttab_hbm, ptab_hbm, out_hbm,
             idx_t, idx_p, ptab_v, b0, b1,
             g0, g1, s0, s1):
    bufs = (b0, b1)
    gs = (g0, g1)
    ss = (s0, s1)

    wid = lax.axis_index("s") * NC + lax.axis_index("c")
    base = pl.multiple_of(wid * WROWS, 8)
    pltpu.sync_copy(tok_hbm.at[pl.ds(base, WROWS)],
                    idx_t.at[pl.ds(0, WROWS)])
    pltpu.sync_copy(posn_hbm.at[pl.ds(base, WROWS)],
                    idx_p.at[pl.ds(0, WROWS)])
    pltpu.sync_copy(ptab_hbm, ptab_v)

    def issue_gather(c, par):
        pltpu.async_copy(
            ttab_hbm.at[idx_t.at[pl.ds(pl.multiple_of(c * CR, 8), CR)]],
            bufs[par], gs[par])

    def wait_gather(par):
        pltpu.make_async_copy(ttab_hbm.at[idx_t.at[pl.ds(0, CR)]],
                              bufs[par], gs[par]).wait()

    def issue_store(c, par):
        pltpu.async_copy(
            bufs[par],
            out_hbm.at[pl.ds(pl.multiple_of(base + c * CR, 8), CR)],
            ss[par])

    def wait_store(par):
        pltpu.make_async_copy(bufs[par], out_hbm.at[pl.ds(0, CR)],
                              ss[par]).wait()

    cols = [jnp.arange(16, dtype=jnp.int32) + 16 * g for g in range(SUB)]

    def compute_chunk(c, par):
        p_vec = idx_p[pl.ds(pl.multiple_of(c * CR, 8), 16)]

        def row(r, carry):
            pb = p_vec.at[jnp.full((16,), r, dtype=jnp.int32)].get(
                mode="promise_in_bounds")
            for q in range(0, SUB, 8):
                pvs = [plsc.load_gather(ptab_v, [pb + cols[g]])
                       for g in range(q, q + 8)]
                for g in range(q, q + 8):
                    plsc.addupdate(bufs[par].at[r, pl.ds(16 * g, 16)],
                                   pvs[g - q])
            return carry

        lax.fori_loop(0, CR, row, 0)

    issue_gather(0, 0)
    issue_gather(1, 1)

    def pair(c2, carry):
        for par in range(2):
            c = 2 * c2 + par
            wait_gather(par)
            compute_chunk(c, par)
            issue_store(c, par)

            def regather():
                wait_store(par)
                issue_gather(c + 2, par)

            pl.when(c < NCH - 2)(regather)
        return carry

    lax.fori_loop(0, NCH // 2, pair, 0)
    wait_store(0)
    wait_store(1)


def kernel(tokens, positions, token_table, pos_table):
    tok = tokens.reshape(ROWS)
    posn = positions.reshape(ROWS) * EMBED
    ptab = pos_table.reshape(MAX_LEN * EMBED)
    mesh = plsc.VectorSubcoreMesh(
        core_axis_name="c", subcore_axis_name="s",
        num_cores=NC, num_subcores=NS)
    out = pl.kernel(
        _sc_body,
        out_type=jax.ShapeDtypeStruct((ROWS, EMBED), jnp.float32),
        mesh=mesh,
        compiler_params=pltpu.CompilerParams(
            use_tc_tiling_on_sc=False, needs_layout_passes=False),
        scratch_types=[
            pltpu.VMEM((WROWS,), jnp.int32),
            pltpu.VMEM((WROWS + 16,), jnp.int32),
            pltpu.VMEM((MAX_LEN * EMBED,), jnp.float32),
            pltpu.VMEM((CR, EMBED), jnp.float32),
            pltpu.VMEM((CR, EMBED), jnp.float32),
            pltpu.SemaphoreType.DMA,
            pltpu.SemaphoreType.DMA,
            pltpu.SemaphoreType.DMA,
            pltpu.SemaphoreType.DMA,
        ],
    )(tok, posn, token_table, ptab)
    return out.reshape(BATCH, MAX_LEN, EMBED)
